# baseline (speedup 1.0000x reference)
import functools

import jax
import jax.numpy as jnp
from jax import lax
from jax.experimental import pallas as pl
from jax.experimental.pallas import tpu as pltpu
from jax.experimental.pallas import tpu_sc as plsc

N_NODES = 10000
N_EDGES = 320000
D = 128
OUT = 64
NUM_GRAPHS = 64
NUM_LAYERS = 3

_NC = 2
_NS = 16
_NW = _NC * _NS
_EPW = N_EDGES // _NW
_CHUNK = 48
_NCHUNK = 209
_NPAD = 10240
_RPT = _NPAD // _NS
_IDX_SHIFT = 14


def _sc_agg_body(h_hbm, cidx_hbm, out_hbm, acc, sem_ci, gsem0, gsem1,
                 ssem0, ssem1):
    def _scoped(cidx_v, rows0_v, rows1_v, rows2_v, src_st, dst_st):
        _sc_agg_scoped(h_hbm, cidx_hbm, out_hbm, cidx_v, rows0_v, rows1_v,
                       rows2_v, src_st, dst_st, acc, sem_ci, gsem0, gsem1,
                       ssem0, ssem1)
    pl.run_scoped(
        _scoped,
        pltpu.VMEM((_NCHUNK, _CHUNK), jnp.int32),
        pltpu.VMEM((_CHUNK, D), jnp.float32),
        pltpu.VMEM((_CHUNK, D), jnp.float32),
        pltpu.VMEM((_CHUNK, D), jnp.float32),
        pltpu.VMEM((6, _CHUNK), jnp.int32),
        pltpu.VMEM((6, _CHUNK), jnp.int32),
    )


def _sc_agg_scoped(h_hbm, cidx_hbm, out_hbm, cidx_v, rows0_v, rows1_v, rows2_v,
                   src_st, dst_st, acc, sem_ci, gsem0, gsem1, ssem0, ssem1):
    c = lax.axis_index("c")
    s = lax.axis_index("s")
    wid = s * _NC + c
    rows = (rows0_v, rows1_v, rows2_v)
    gsems = (gsem0, gsem0, gsem0)

    pltpu.async_copy(cidx_hbm.at[wid], cidx_v, sem_ci)

    def _zrow(i, _):
        for j in range(D // 16):
            rows0_v[i, pl.ds(j * 16, 16)] = jnp.zeros((16,), jnp.float32)
        return 0
    lax.fori_loop(0, _CHUNK, _zrow, 0)
    for r in range(_RPT // 40):
        pltpu.sync_copy(rows0_v.at[pl.ds(0, 40)],
                        acc.at[pl.ds(s * _RPT + r * 40, 40)])
    pltpu.make_async_copy(cidx_hbm.at[wid], cidx_v, sem_ci).wait()
    plsc.subcore_barrier()

    def _unpack_issue(i, q):
        b = q % 3
        for j in range(_CHUNK // 16):
            w = cidx_v[i, pl.ds(j * 16, 16)]
            src_st[q, pl.ds(j * 16, 16)] = lax.shift_right_logical(w, _IDX_SHIFT)
            dst_st[q, pl.ds(j * 16, 16)] = w & ((1 << _IDX_SHIFT) - 1)
        pltpu.async_copy(h_hbm.at[src_st.at[q]], rows[b], gsems[b])

    def _wait_gather(q):
        b = q % 3
        pltpu.make_async_copy(h_hbm.at[src_st.at[q]], rows[b], gsems[b]).wait()

    for i in range(3):
        _unpack_issue(i, i)

    _NHEX = (_NCHUNK - 3) // 6
    _TAIL = 3 + 6 * _NHEX

    def _hex(g, _):
        for k in range(6):
            i = 3 + 6 * g + k
            q = (3 + k) % 6
            _wait_gather((q + 3) % 6)
            _unpack_issue(i, q)
        return 0
    lax.fori_loop(0, _NHEX, _hex, 0)
    for i in range(_TAIL, _NCHUNK):
        q = i % 6
        _wait_gather((q + 3) % 6)
        _unpack_issue(i, q)
    for i in range(_NCHUNK - 3, _NCHUNK):
        _wait_gather(i % 6)
    plsc.subcore_barrier()

    pltpu.sync_copy(acc.at[pl.ds(s * _RPT, _RPT)],
                    out_hbm.at[c, pl.ds(s * _RPT, _RPT)])


@functools.partial(
    pl.kernel,
    out_type=jax.ShapeDtypeStruct((_NC, _NPAD, D), jnp.float32),
    mesh=plsc.VectorSubcoreMesh(core_axis_name="c", subcore_axis_name="s",
                                num_cores=_NC, num_subcores=_NS),
    scratch_types=[
        pltpu.VMEM_SHARED((_NPAD, D), jnp.float32),
        pltpu.SemaphoreType.DMA,
        pltpu.SemaphoreType.DMA,
        pltpu.SemaphoreType.DMA,
        pltpu.SemaphoreType.DMA,
        pltpu.SemaphoreType.DMA,
    ],
)
def _sc_agg(h_hbm, cidx_hbm, out_hbm, acc, sem_ci, gsem0, gsem1, ssem0, ssem1):
    _sc_agg_body(h_hbm, cidx_hbm, out_hbm, acc, sem_ci, gsem0, gsem1,
                 ssem0, ssem1)


def _mlp_body(h_ref, agg_ref, eps_ref, w1_ref, b1_ref, g1_ref, be1_ref,
              w2_ref, b2_ref, g2_ref, be2_ref, out_ref):
    z = ((1.0 + eps_ref[...]) * h_ref[...]
         + agg_ref[0, :N_NODES, :] + agg_ref[1, :N_NODES, :])
    z = jnp.dot(z, w1_ref[...], preferred_element_type=jnp.float32) + b1_ref[...]
    mu = jnp.mean(z, axis=0, keepdims=True)
    zc = z - mu
    var = jnp.mean(zc * zc, axis=0, keepdims=True)
    z = zc * lax.rsqrt(var + 1e-5) * g1_ref[...] + be1_ref[...]
    z = jnp.maximum(z, 0.0)
    z = jnp.dot(z, w2_ref[...], preferred_element_type=jnp.float32) + b2_ref[...]
    mu = jnp.mean(z, axis=0, keepdims=True)
    zc = z - mu
    var = jnp.mean(zc * zc, axis=0, keepdims=True)
    z = zc * lax.rsqrt(var + 1e-5) * g2_ref[...] + be2_ref[...]
    out_ref[...] = jnp.maximum(z, 0.0)


_mlp_call = pl.pallas_call(
    _mlp_body,
    out_shape=jax.ShapeDtypeStruct((N_NODES, D), jnp.float32),
)


def _pool_body(h1_ref, h2_ref, h3_ref, batch_ref, wos_ref, bos_ref, out_ref):
    gid = lax.broadcasted_iota(jnp.int32, (NUM_GRAPHS, N_NODES), 0)
    sel = (batch_ref[...] == gid).astype(jnp.float32)
    acc = jnp.zeros((NUM_GRAPHS, OUT), jnp.float32)
    for l, h_ref in enumerate((h1_ref, h2_ref, h3_ref)):
        pooled = jnp.dot(sel, h_ref[...], preferred_element_type=jnp.float32)
        acc = acc + jnp.dot(pooled, wos_ref[l],
                            preferred_element_type=jnp.float32) + bos_ref[l]
    out_ref[...] = acc


_pool_call = pl.pallas_call(
    _pool_body,
    out_shape=jax.ShapeDtypeStruct((NUM_GRAPHS, OUT), jnp.float32),
)


def kernel(x, edge_index, batch, W1s, b1s, g1s, be1s, W2s, b2s, g2s, be2s,
           eps, Wos, bos):
    ei = edge_index.astype(jnp.int32)
    cidx = (ei[0] << _IDX_SHIFT) | ei[1]
    pad = _NW * _NCHUNK * _CHUNK - N_EDGES
    cidx = jnp.concatenate([cidx, jnp.full((pad,), N_NODES, jnp.int32)])
    cidx = cidx.reshape(_NW, _NCHUNK, _CHUNK)
    h = x
    hs = []
    for l in range(NUM_LAYERS):
        agg = _sc_agg(h, cidx)
        h = _mlp_call(h, agg, eps[l].reshape(1, 1),
                      W1s[l], b1s[l].reshape(1, D), g1s[l].reshape(1, D),
                      be1s[l].reshape(1, D),
                      W2s[l], b2s[l].reshape(1, D), g2s[l].reshape(1, D),
                      be2s[l].reshape(1, D))
        hs.append(h)
    return _pool_call(hs[0], hs[1], hs[2], batch.astype(jnp.int32).reshape(1, N_NODES),
                      Wos, bos.reshape(NUM_LAYERS, 1, OUT))


# speedup vs baseline: 1.1426x; 1.1426x over previous
import functools

import jax
import jax.numpy as jnp
from jax import lax
from jax.experimental import pallas as pl
from jax.experimental.pallas import tpu as pltpu
from jax.experimental.pallas import tpu_sc as plsc

N_NODES = 10000
N_EDGES = 320000
D = 128
OUT = 64
NUM_GRAPHS = 64
NUM_LAYERS = 3

_NC = 2
_NS = 16
_NW = _NC * _NS
_EPW = N_EDGES // _NW
_CHUNK = 80
_NCHUNK = _EPW // _CHUNK
_NPAD = 10240
_RPT = _NPAD // _NS
_IDX_SHIFT = 14


def _sc_agg_body(h_hbm, cidx_hbm, out_hbm, acc, sem_ci, gsem0, gsem1,
                 ssem0, ssem1):
    def _scoped(cidx_v, rows0_v, rows1_v, src_st, dst_st):
        _sc_agg_scoped(h_hbm, cidx_hbm, out_hbm, cidx_v, rows0_v, rows1_v,
                       src_st, dst_st, acc, sem_ci, gsem0, gsem1, ssem0, ssem1)
    pl.run_scoped(
        _scoped,
        pltpu.VMEM((_NCHUNK, _CHUNK), jnp.int32),
        pltpu.VMEM((_CHUNK, D), jnp.float32),
        pltpu.VMEM((_CHUNK, D), jnp.float32),
        pltpu.VMEM((4, _CHUNK), jnp.int32),
        pltpu.VMEM((4, _CHUNK), jnp.int32),
    )


def _sc_agg_scoped(h_hbm, cidx_hbm, out_hbm, cidx_v, rows0_v, rows1_v,
                   src_st, dst_st, acc, sem_ci, gsem0, gsem1, ssem0, ssem1):
    c = lax.axis_index("c")
    s = lax.axis_index("s")
    wid = s * _NC + c
    rows = (rows0_v, rows1_v)
    gsems = (gsem0, gsem1)
    ssems = (ssem0, ssem1)

    pltpu.async_copy(cidx_hbm.at[wid], cidx_v, sem_ci)

    def _zrow(i, _):
        for j in range(D // 16):
            rows0_v[i, pl.ds(j * 16, 16)] = jnp.zeros((16,), jnp.float32)
        return 0
    lax.fori_loop(0, _CHUNK, _zrow, 0)
    for r in range(_RPT // _CHUNK):
        pltpu.sync_copy(rows0_v, acc.at[pl.ds(s * _RPT + r * _CHUNK, _CHUNK)])
    pltpu.make_async_copy(cidx_hbm.at[wid], cidx_v, sem_ci).wait()
    plsc.subcore_barrier()

    def _unpack_issue(i, q):
        b = q % 2
        for j in range(_CHUNK // 16):
            w = cidx_v[i, pl.ds(j * 16, 16)]
            src_st[q, pl.ds(j * 16, 16)] = lax.shift_right_logical(w, _IDX_SHIFT)
            dst_st[q, pl.ds(j * 16, 16)] = w & ((1 << _IDX_SHIFT) - 1)
        pltpu.async_copy(h_hbm.at[src_st.at[q]], rows[b], gsems[b])

    def _scatter(q):
        b = q % 2
        pltpu.make_async_copy(h_hbm.at[src_st.at[q]], rows[b], gsems[b]).wait()
        pltpu.async_copy(rows[b], acc.at[dst_st.at[q]], ssems[b], add=True)

    def _wait_scatter(q):
        b = q % 2
        pltpu.make_async_copy(rows[b], acc.at[dst_st.at[q]], ssems[b]).wait()

    def _step(i, q):
        @pl.when(i >= 2)
        def _():
            _wait_scatter((q + 2) % 4)
        _unpack_issue(i, q)

        @pl.when(i >= 1)
        def _():
            _scatter((q + 3) % 4)

    def _quad(g, _):
        for k in range(4):
            _step(4 * g + k, k)
        return 0
    lax.fori_loop(0, _NCHUNK // 4, _quad, 0)
    for k in range(_NCHUNK % 4):
        _step((_NCHUNK // 4) * 4 + k, k)
    _scatter((_NCHUNK + 3) % 4)
    _wait_scatter((_NCHUNK + 2) % 4)
    _wait_scatter((_NCHUNK + 3) % 4)
    plsc.subcore_barrier()

    pltpu.sync_copy(acc.at[pl.ds(s * _RPT, _RPT)],
                    out_hbm.at[c, pl.ds(s * _RPT, _RPT)])


@functools.partial(
    pl.kernel,
    out_type=jax.ShapeDtypeStruct((_NC, _NPAD, D), jnp.float32),
    mesh=plsc.VectorSubcoreMesh(core_axis_name="c", subcore_axis_name="s",
                                num_cores=_NC, num_subcores=_NS),
    scratch_types=[
        pltpu.VMEM_SHARED((_NPAD, D), jnp.float32),
        pltpu.SemaphoreType.DMA,
        pltpu.SemaphoreType.DMA,
        pltpu.SemaphoreType.DMA,
        pltpu.SemaphoreType.DMA,
        pltpu.SemaphoreType.DMA,
    ],
)
def _sc_agg(h_hbm, cidx_hbm, out_hbm, acc, sem_ci, gsem0, gsem1, ssem0, ssem1):
    _sc_agg_body(h_hbm, cidx_hbm, out_hbm, acc, sem_ci, gsem0, gsem1,
                 ssem0, ssem1)


def _mlp_body(h_ref, agg_ref, batch_ref, eps_ref, w1_ref, b1_ref, g1_ref,
              be1_ref, w2_ref, b2_ref, g2_ref, be2_ref, wo_ref, bo_ref,
              out_ref, score_ref):
    z = ((1.0 + eps_ref[...]) * h_ref[...]
         + agg_ref[0, :N_NODES, :] + agg_ref[1, :N_NODES, :])
    z = jnp.dot(z, w1_ref[...], preferred_element_type=jnp.float32) + b1_ref[...]
    mu = jnp.mean(z, axis=0, keepdims=True)
    zc = z - mu
    var = jnp.mean(zc * zc, axis=0, keepdims=True)
    z = zc * lax.rsqrt(var + 1e-5) * g1_ref[...] + be1_ref[...]
    z = jnp.maximum(z, 0.0)
    z = jnp.dot(z, w2_ref[...], preferred_element_type=jnp.float32) + b2_ref[...]
    mu = jnp.mean(z, axis=0, keepdims=True)
    zc = z - mu
    var = jnp.mean(zc * zc, axis=0, keepdims=True)
    z = zc * lax.rsqrt(var + 1e-5) * g2_ref[...] + be2_ref[...]
    z = jnp.maximum(z, 0.0)
    out_ref[...] = z
    gid = lax.broadcasted_iota(jnp.int32, (NUM_GRAPHS, N_NODES), 0)
    sel = (batch_ref[...] == gid).astype(jnp.float32)
    pooled = jnp.dot(sel, z, preferred_element_type=jnp.float32)
    score_ref[...] = jnp.dot(pooled, wo_ref[...],
                             preferred_element_type=jnp.float32) + bo_ref[...]


_mlp_call = pl.pallas_call(
    _mlp_body,
    out_shape=(jax.ShapeDtypeStruct((N_NODES, D), jnp.float32),
               jax.ShapeDtypeStruct((NUM_GRAPHS, OUT), jnp.float32)),
)


def kernel(x, edge_index, batch, W1s, b1s, g1s, be1s, W2s, b2s, g2s, be2s,
           eps, Wos, bos):
    ei = edge_index.astype(jnp.int32)
    cidx = ((ei[0] << _IDX_SHIFT) | ei[1]).reshape(_NW, _NCHUNK, _CHUNK)
    batch2 = batch.astype(jnp.int32).reshape(1, N_NODES)
    h = x
    score = jnp.zeros((NUM_GRAPHS, OUT), jnp.float32)
    for l in range(NUM_LAYERS):
        agg = _sc_agg(h, cidx)
        h, score_l = _mlp_call(h, agg, batch2, eps[l].reshape(1, 1),
                               W1s[l], b1s[l].reshape(1, D),
                               g1s[l].reshape(1, D), be1s[l].reshape(1, D),
                               W2s[l], b2s[l].reshape(1, D),
                               g2s[l].reshape(1, D), be2s[l].reshape(1, D),
                               Wos[l], bos[l].reshape(1, OUT))
        score = score + score_l
    return score


# speedup vs baseline: 1.3707x; 1.1996x over previous
import functools

import jax
import jax.numpy as jnp
from jax import lax
from jax.experimental import pallas as pl
from jax.experimental.pallas import tpu as pltpu
from jax.experimental.pallas import tpu_sc as plsc

N_NODES = 10000
N_EDGES = 320000
D = 128
OUT = 64
NUM_GRAPHS = 64
NUM_LAYERS = 3

_NC = 2
_NS = 16
_NW = _NC * _NS
_EPW = N_EDGES // _NW
_CHUNK = 80
_NCHUNK = _EPW // _CHUNK
_NPAD = 10240
_RPT = _NPAD // _NS
_IDX_SHIFT = 14


def _sc_agg_body(h_hbm, srcf_hbm, dstf_hbm, out_hbm, acc,
                 sem_si, sem_di, gsem, sem_x):
    def _scoped(rows0_v, rows1_v, rows2_v, src_st, dst_st):
        _sc_agg_scoped(h_hbm, srcf_hbm, dstf_hbm, out_hbm, rows0_v, rows1_v,
                       rows2_v, src_st, dst_st, acc, sem_si, sem_di, gsem)
    pl.run_scoped(
        _scoped,
        pltpu.VMEM((_CHUNK, D), jnp.float32),
        pltpu.VMEM((_CHUNK, D), jnp.float32),
        pltpu.VMEM((_CHUNK, D), jnp.float32),
        pltpu.VMEM((6, _CHUNK), jnp.int32),
        pltpu.VMEM((6, _CHUNK), jnp.int32),
    )


def _sc_agg_scoped(h_hbm, srcf_hbm, dstf_hbm, out_hbm, rows0_v, rows1_v,
                   rows2_v, src_st, dst_st, acc, sem_si, sem_di, gsem):
    c = lax.axis_index("c")
    s = lax.axis_index("s")
    wid = s * _NC + c
    base0 = wid * _EPW
    rows = (rows0_v, rows1_v, rows2_v)

    def _idx_issue(i, q):
        base = base0 + i * _CHUNK
        pltpu.async_copy(srcf_hbm.at[pl.ds(base, _CHUNK)], src_st.at[q], sem_si)
        pltpu.async_copy(dstf_hbm.at[pl.ds(base, _CHUNK)], dst_st.at[q], sem_di)

    def _idx_wait(i, q):
        base = base0 + i * _CHUNK
        pltpu.make_async_copy(srcf_hbm.at[pl.ds(base, _CHUNK)], src_st.at[q],
                              sem_si).wait()
        pltpu.make_async_copy(dstf_hbm.at[pl.ds(base, _CHUNK)], dst_st.at[q],
                              sem_di).wait()

    def _gather(i, q):
        pltpu.async_copy(h_hbm.at[src_st.at[q]], rows[q % 3], gsem)

    def _gwait_scatter(j, qj):
        pltpu.make_async_copy(h_hbm.at[src_st.at[qj]], rows[qj % 3],
                              gsem).wait()
        pltpu.sync_copy(rows[qj % 3], acc.at[dst_st.at[qj]], add=True)

    for i in range(3):
        _idx_issue(i, i)

    def _zrow(i, _):
        for j in range(D // 16):
            rows0_v[i, pl.ds(j * 16, 16)] = jnp.zeros((16,), jnp.float32)
        return 0
    lax.fori_loop(0, _CHUNK, _zrow, 0)
    for r in range(_RPT // _CHUNK):
        pltpu.sync_copy(rows0_v, acc.at[pl.ds(s * _RPT + r * _CHUNK, _CHUNK)])
    plsc.subcore_barrier()

    for i in range(2):
        _idx_wait(i, i)
        _gather(i, i)
        _idx_issue(i + 3, i + 3)

    def _steady(i, q):
        _idx_wait(i, q)
        _gather(i, q)
        if isinstance(i, int) and i + 3 >= _NCHUNK:
            pass
        else:
            _idx_issue(i + 3, (q + 3) % 6)
        _gwait_scatter(i - 2, (q + 4) % 6)

    def _hex(g, _):
        for k in range(6):
            _steady(2 + 6 * g + k, (2 + k) % 6)
        return 0
    _NHEX = (_NCHUNK - 2 - 3) // 6
    lax.fori_loop(0, _NHEX, _hex, 0)
    for i in range(2 + 6 * _NHEX, _NCHUNK):
        _steady(i, i % 6)
    _gwait_scatter(_NCHUNK - 2, (_NCHUNK - 2) % 6)
    _gwait_scatter(_NCHUNK - 1, (_NCHUNK - 1) % 6)
    plsc.subcore_barrier()

    pltpu.sync_copy(acc.at[pl.ds(s * _RPT, _RPT)],
                    out_hbm.at[c, pl.ds(s * _RPT, _RPT)])


@functools.partial(
    pl.kernel,
    out_type=jax.ShapeDtypeStruct((_NC, _NPAD, D), jnp.float32),
    mesh=plsc.VectorSubcoreMesh(core_axis_name="c", subcore_axis_name="s",
                                num_cores=_NC, num_subcores=_NS),
    scratch_types=[
        pltpu.VMEM_SHARED((_NPAD, D), jnp.float32),
        pltpu.SemaphoreType.DMA,
        pltpu.SemaphoreType.DMA,
        pltpu.SemaphoreType.DMA,
        pltpu.SemaphoreType.DMA,
    ],
)
def _sc_agg(h_hbm, srcf_hbm, dstf_hbm, out_hbm, acc, sem_si, sem_di, gsem,
            sem_x):
    _sc_agg_body(h_hbm, srcf_hbm, dstf_hbm, out_hbm, acc,
                 sem_si, sem_di, gsem, sem_x)


def _mlp_body(h_ref, agg_ref, batch_ref, eps_ref, w1_ref, b1_ref, g1_ref,
              be1_ref, w2_ref, b2_ref, g2_ref, be2_ref, wo_ref, bo_ref,
              out_ref, score_ref):
    z = ((1.0 + eps_ref[...]) * h_ref[...]
         + agg_ref[0, :N_NODES, :] + agg_ref[1, :N_NODES, :])
    z = jnp.dot(z, w1_ref[...], preferred_element_type=jnp.float32) + b1_ref[...]
    mu = jnp.mean(z, axis=0, keepdims=True)
    zc = z - mu
    var = jnp.mean(zc * zc, axis=0, keepdims=True)
    z = zc * lax.rsqrt(var + 1e-5) * g1_ref[...] + be1_ref[...]
    z = jnp.maximum(z, 0.0)
    z = jnp.dot(z, w2_ref[...], preferred_element_type=jnp.float32) + b2_ref[...]
    mu = jnp.mean(z, axis=0, keepdims=True)
    zc = z - mu
    var = jnp.mean(zc * zc, axis=0, keepdims=True)
    z = zc * lax.rsqrt(var + 1e-5) * g2_ref[...] + be2_ref[...]
    z = jnp.maximum(z, 0.0)
    out_ref[...] = z
    gid = lax.broadcasted_iota(jnp.int32, (NUM_GRAPHS, N_NODES), 0)
    sel = (batch_ref[...] == gid).astype(jnp.float32)
    pooled = jnp.dot(sel, z, preferred_element_type=jnp.float32)
    score_ref[...] = jnp.dot(pooled, wo_ref[...],
                             preferred_element_type=jnp.float32) + bo_ref[...]


_mlp_call = pl.pallas_call(
    _mlp_body,
    out_shape=(jax.ShapeDtypeStruct((N_NODES, D), jnp.float32),
               jax.ShapeDtypeStruct((NUM_GRAPHS, OUT), jnp.float32)),
)


def kernel(x, edge_index, batch, W1s, b1s, g1s, be1s, W2s, b2s, g2s, be2s,
           eps, Wos, bos):
    ei = edge_index.astype(jnp.int32)
    srcf, dstf = ei[0], ei[1]
    batch2 = batch.astype(jnp.int32).reshape(1, N_NODES)
    h = x
    score = jnp.zeros((NUM_GRAPHS, OUT), jnp.float32)
    for l in range(NUM_LAYERS):
        agg = _sc_agg(h, srcf, dstf)
        h, score_l = _mlp_call(h, agg, batch2, eps[l].reshape(1, 1),
                               W1s[l], b1s[l].reshape(1, D),
                               g1s[l].reshape(1, D), be1s[l].reshape(1, D),
                               W2s[l], b2s[l].reshape(1, D),
                               g2s[l].reshape(1, D), be2s[l].reshape(1, D),
                               Wos[l], bos[l].reshape(1, OUT))
        score = score + score_l
    return score
